# Initial kernel scaffold; baseline (speedup 1.0000x reference)
import jax
import jax.numpy as jnp
from jax.experimental import pallas as pl


def kernel(nodes, edges, W1, b1, W2, b2, senders, receivers):
    raise NotImplementedError("write your pallas kernel here")


# speedup vs baseline: 1.9861x; 1.9861x over previous
import functools

import jax
import jax.numpy as jnp
from jax import lax
from jax.experimental import pallas as pl
from jax.experimental.pallas import tpu as pltpu
from jax.experimental.pallas import tpu_sc as plsc

_NC, _NS = 2, 16
_NW = _NC * _NS
_CH = 80
_D_NODE = 128
_BE = 2560


def _gather_body(nodes_hbm, s_hbm, r_hbm, out_s, out_r,
                 idx_s, idx_r, rows_s, rows_r, sem_s, sem_r):
    e_total = s_hbm.shape[0]
    b_per_w = e_total // _NW
    nch = b_per_w // _CH
    wid = lax.axis_index("s") * _NC + lax.axis_index("c")
    base = wid * b_per_w

    def body(i, carry):
        off = pl.multiple_of(base + i * _CH, 8)
        pltpu.sync_copy(s_hbm.at[pl.ds(off, _CH)], idx_s)
        pltpu.sync_copy(r_hbm.at[pl.ds(off, _CH)], idx_r)
        cs = pltpu.async_copy(nodes_hbm.at[idx_s], rows_s, sem_s)
        cr = pltpu.async_copy(nodes_hbm.at[idx_r], rows_r, sem_r)
        cs.wait()
        cr.wait()
        pltpu.sync_copy(rows_s, out_s.at[pl.ds(off, _CH)])
        pltpu.sync_copy(rows_r, out_r.at[pl.ds(off, _CH)])
        return carry

    lax.fori_loop(0, nch, body, 0)


def _sc_gather(nodes, senders, receivers):
    e_total = senders.shape[0]
    mesh = plsc.VectorSubcoreMesh(core_axis_name="c", subcore_axis_name="s")
    f = pl.kernel(
        _gather_body,
        out_type=[
            jax.ShapeDtypeStruct((e_total, _D_NODE), jnp.float32),
            jax.ShapeDtypeStruct((e_total, _D_NODE), jnp.float32),
        ],
        mesh=mesh,
        scratch_types=[
            pltpu.VMEM((_CH,), jnp.int32),
            pltpu.VMEM((_CH,), jnp.int32),
            pltpu.VMEM((_CH, _D_NODE), jnp.float32),
            pltpu.VMEM((_CH, _D_NODE), jnp.float32),
            pltpu.SemaphoreType.DMA,
            pltpu.SemaphoreType.DMA,
        ],
    )
    return f(nodes, senders, receivers)


def _mlp_body(sg_ref, rg_ref, ef_ref, w1a_ref, w1b_ref, w1c_ref, b1_ref,
              w2_ref, b2_ref, out_ref):
    x = (jnp.dot(sg_ref[...], w1a_ref[...], preferred_element_type=jnp.float32)
         + jnp.dot(rg_ref[...], w1b_ref[...], preferred_element_type=jnp.float32)
         + jnp.dot(ef_ref[...], w1c_ref[...], preferred_element_type=jnp.float32)
         + b1_ref[...])
    h = 1.0 / (1.0 + jnp.exp(-x))
    out_ref[...] = (jnp.dot(h, w2_ref[...], preferred_element_type=jnp.float32)
                    + b2_ref[...])


def _tc_mlp(sg, rg, edges, w1a, w1b, w1c, b1, w2, b2):
    e_total, d_edge = edges.shape
    d_hidden = w2.shape[0]
    d_out = w2.shape[1]
    grid = (e_total // _BE,)
    return pl.pallas_call(
        _mlp_body,
        grid=grid,
        in_specs=[
            pl.BlockSpec((_BE, _D_NODE), lambda i: (i, 0)),
            pl.BlockSpec((_BE, _D_NODE), lambda i: (i, 0)),
            pl.BlockSpec((_BE, d_edge), lambda i: (i, 0)),
            pl.BlockSpec((_D_NODE, d_hidden), lambda i: (0, 0)),
            pl.BlockSpec((_D_NODE, d_hidden), lambda i: (0, 0)),
            pl.BlockSpec((d_edge, d_hidden), lambda i: (0, 0)),
            pl.BlockSpec((1, d_hidden), lambda i: (0, 0)),
            pl.BlockSpec((d_hidden, d_out), lambda i: (0, 0)),
            pl.BlockSpec((1, d_out), lambda i: (0, 0)),
        ],
        out_specs=pl.BlockSpec((_BE, d_out), lambda i: (i, 0)),
        out_shape=jax.ShapeDtypeStruct((e_total, d_out), jnp.float32),
    )(sg, rg, edges, w1a, w1b, w1c, b1, w2, b2)


@jax.jit
def kernel(nodes, edges, W1, b1, W2, b2, senders, receivers):
    senders = senders.astype(jnp.int32)
    receivers = receivers.astype(jnp.int32)
    e_total = senders.shape[0]
    assert e_total % (_NW * _CH) == 0 and e_total % _BE == 0

    sg, rg = _sc_gather(nodes, senders, receivers)

    d_node = nodes.shape[1]
    w1t = W1.T
    w1a = w1t[:d_node]
    w1b = w1t[d_node:2 * d_node]
    w1c = w1t[2 * d_node:]
    return _tc_mlp(sg, rg, edges, w1a, w1b, w1c,
                   b1.reshape(1, -1), W2.T, b2.reshape(1, -1))


# speedup vs baseline: 2.2640x; 1.1399x over previous
import functools

import jax
import jax.numpy as jnp
from jax import lax
from jax.experimental import pallas as pl
from jax.experimental.pallas import tpu as pltpu
from jax.experimental.pallas import tpu_sc as plsc

_NC, _NS = 2, 16
_NW = _NC * _NS
_CH = 40
_D_NODE = 128
_BE = 2560


def _gather_body(nodes_hbm, s_hbm, r_hbm, out_s, out_r,
                 idx_s, idx_r, bufs, sems_g, sems_w):
    e_total = s_hbm.shape[0]
    b_per_w = e_total // _NW
    nch = b_per_w // _CH
    wid = lax.axis_index("s") * _NC + lax.axis_index("c")
    base = wid * b_per_w

    pltpu.sync_copy(s_hbm.at[pl.ds(base, b_per_w)], idx_s)
    pltpu.sync_copy(r_hbm.at[pl.ds(base, b_per_w)], idx_r)
    idx = (idx_s, idx_r)
    out = (out_s, out_r)

    def fire_gather(ring, c):
        return [
            pltpu.async_copy(
                nodes_hbm.at[idx[st].at[pl.ds(c * _CH, _CH)]],
                bufs[ring][st], sems_g[ring][st])
            for st in (0, 1)
        ]

    def fire_write(ring, c):
        off = pl.multiple_of(base + c * _CH, 8)
        for st in (0, 1):
            pltpu.async_copy(bufs[ring][st], out[st].at[pl.ds(off, _CH)],
                             sems_w[ring][st])

    def wait_write(ring):
        for st in (0, 1):
            pltpu.make_async_copy(bufs[ring][st],
                                  out[st].at[pl.ds(0, _CH)],
                                  sems_w[ring][st]).wait()

    ga = fire_gather(0, 0)
    gb = fire_gather(1, 1)
    for d in ga:
        d.wait()
    fire_write(0, 0)
    for d in gb:
        d.wait()
    fire_write(1, 1)

    def body(t, carry):
        c0 = t * 2
        c1 = c0 + 1
        wait_write(0)
        ga = fire_gather(0, c0)
        wait_write(1)
        gb = fire_gather(1, c1)
        for d in ga:
            d.wait()
        fire_write(0, c0)
        for d in gb:
            d.wait()
        fire_write(1, c1)
        return carry

    lax.fori_loop(1, nch // 2, body, 0)
    wait_write(0)
    wait_write(1)


def _sc_gather(nodes, senders, receivers):
    e_total = senders.shape[0]
    b_per_w = e_total // _NW
    mesh = plsc.VectorSubcoreMesh(core_axis_name="c", subcore_axis_name="s")
    f = pl.kernel(
        _gather_body,
        out_type=[
            jax.ShapeDtypeStruct((e_total, _D_NODE), jnp.float32),
            jax.ShapeDtypeStruct((e_total, _D_NODE), jnp.float32),
        ],
        mesh=mesh,
        scratch_types=[
            pltpu.VMEM((b_per_w,), jnp.int32),
            pltpu.VMEM((b_per_w,), jnp.int32),
            [[pltpu.VMEM((_CH, _D_NODE), jnp.float32) for _ in range(2)]
             for _ in range(2)],
            [[pltpu.SemaphoreType.DMA for _ in range(2)] for _ in range(2)],
            [[pltpu.SemaphoreType.DMA for _ in range(2)] for _ in range(2)],
        ],
    )
    return f(nodes, senders, receivers)


def _mlp_body(sg_ref, rg_ref, ef_ref, w1a_ref, w1b_ref, w1c_ref, b1_ref,
              w2_ref, b2_ref, out_ref):
    x = (jnp.dot(sg_ref[...], w1a_ref[...], preferred_element_type=jnp.float32)
         + jnp.dot(rg_ref[...], w1b_ref[...], preferred_element_type=jnp.float32)
         + jnp.dot(ef_ref[...], w1c_ref[...], preferred_element_type=jnp.float32)
         + b1_ref[...])
    h = 1.0 / (1.0 + jnp.exp(-x))
    out_ref[...] = (jnp.dot(h, w2_ref[...], preferred_element_type=jnp.float32)
                    + b2_ref[...])


def _tc_mlp(sg, rg, edges, w1a, w1b, w1c, b1, w2, b2):
    e_total, d_edge = edges.shape
    d_hidden = w2.shape[0]
    d_out = w2.shape[1]
    grid = (e_total // _BE,)
    return pl.pallas_call(
        _mlp_body,
        grid=grid,
        in_specs=[
            pl.BlockSpec((_BE, _D_NODE), lambda i: (i, 0)),
            pl.BlockSpec((_BE, _D_NODE), lambda i: (i, 0)),
            pl.BlockSpec((_BE, d_edge), lambda i: (i, 0)),
            pl.BlockSpec((_D_NODE, d_hidden), lambda i: (0, 0)),
            pl.BlockSpec((_D_NODE, d_hidden), lambda i: (0, 0)),
            pl.BlockSpec((d_edge, d_hidden), lambda i: (0, 0)),
            pl.BlockSpec((1, d_hidden), lambda i: (0, 0)),
            pl.BlockSpec((d_hidden, d_out), lambda i: (0, 0)),
            pl.BlockSpec((1, d_out), lambda i: (0, 0)),
        ],
        out_specs=pl.BlockSpec((_BE, d_out), lambda i: (i, 0)),
        out_shape=jax.ShapeDtypeStruct((e_total, d_out), jnp.float32),
    )(sg, rg, edges, w1a, w1b, w1c, b1, w2, b2)


@jax.jit
def kernel(nodes, edges, W1, b1, W2, b2, senders, receivers):
    senders = senders.astype(jnp.int32)
    receivers = receivers.astype(jnp.int32)
    e_total = senders.shape[0]
    assert e_total % (_NW * _CH) == 0 and e_total % _BE == 0

    sg, rg = _sc_gather(nodes, senders, receivers)

    d_node = nodes.shape[1]
    w1t = W1.T
    w1a = w1t[:d_node]
    w1b = w1t[d_node:2 * d_node]
    w1c = w1t[2 * d_node:]
    return _tc_mlp(sg, rg, edges, w1a, w1b, w1c,
                   b1.reshape(1, -1), W2.T, b2.reshape(1, -1))


# speedup vs baseline: 2.3453x; 1.0359x over previous
import functools

import jax
import jax.numpy as jnp
from jax import lax
from jax.experimental import pallas as pl
from jax.experimental.pallas import tpu as pltpu
from jax.experimental.pallas import tpu_sc as plsc

_NC, _NS = 2, 16
_NW = _NC * _NS
_CH = 40
_D_NODE = 128
_BE = 2560


def _gather_body(nodes_hbm, s_hbm, r_hbm, out_s, out_r,
                 idx_s, idx_r, bufs, sems_g, sems_w):
    e_total = s_hbm.shape[0]
    b_per_w = e_total // _NW
    nch = b_per_w // _CH
    wid = lax.axis_index("s") * _NC + lax.axis_index("c")
    base = wid * b_per_w

    pltpu.sync_copy(s_hbm.at[pl.ds(base, b_per_w)], idx_s)
    pltpu.sync_copy(r_hbm.at[pl.ds(base, b_per_w)], idx_r)
    idx = (idx_s, idx_r)
    out = (out_s, out_r)

    def fire_gather(ring, c):
        return [
            pltpu.async_copy(
                nodes_hbm.at[idx[st].at[pl.ds(c * _CH, _CH)]],
                bufs[ring][st], sems_g[ring][st])
            for st in (0, 1)
        ]

    def fire_write(ring, c):
        off = pl.multiple_of(base + c * _CH, 8)
        for st in (0, 1):
            pltpu.async_copy(bufs[ring][st], out[st].at[pl.ds(off, _CH)],
                             sems_w[ring][st])

    def wait_write(ring):
        for st in (0, 1):
            pltpu.make_async_copy(bufs[ring][st],
                                  out[st].at[pl.ds(0, _CH)],
                                  sems_w[ring][st]).wait()

    ga = fire_gather(0, 0)
    gb = fire_gather(1, 1)
    for d in ga:
        d.wait()
    fire_write(0, 0)
    for d in gb:
        d.wait()
    fire_write(1, 1)

    def body(t, carry):
        c0 = t * 2
        c1 = c0 + 1
        wait_write(0)
        ga = fire_gather(0, c0)
        wait_write(1)
        gb = fire_gather(1, c1)
        for d in ga:
            d.wait()
        fire_write(0, c0)
        for d in gb:
            d.wait()
        fire_write(1, c1)
        return carry

    lax.fori_loop(1, nch // 2, body, 0)
    wait_write(0)
    wait_write(1)


def _sc_gather(nodes, senders, receivers):
    e_total = senders.shape[0]
    b_per_w = e_total // _NW
    mesh = plsc.VectorSubcoreMesh(core_axis_name="c", subcore_axis_name="s")
    f = pl.kernel(
        _gather_body,
        out_type=[
            jax.ShapeDtypeStruct((e_total, _D_NODE), jnp.float32),
            jax.ShapeDtypeStruct((e_total, _D_NODE), jnp.float32),
        ],
        mesh=mesh,
        scratch_types=[
            pltpu.VMEM((b_per_w,), jnp.int32),
            pltpu.VMEM((b_per_w,), jnp.int32),
            [[pltpu.VMEM((_CH, _D_NODE), jnp.float32) for _ in range(2)]
             for _ in range(2)],
            [[pltpu.SemaphoreType.DMA for _ in range(2)] for _ in range(2)],
            [[pltpu.SemaphoreType.DMA for _ in range(2)] for _ in range(2)],
        ],
    )
    return f(nodes, senders, receivers)


def _mlp_body(sg_ref, rg_ref, ef_ref, w1a_ref, w1b_ref, w1c_ref, b1_ref,
              w2_ref, b2_ref, out_ref):
    x = (jnp.dot(sg_ref[...], w1a_ref[...], preferred_element_type=jnp.float32)
         + jnp.dot(rg_ref[...], w1b_ref[...], preferred_element_type=jnp.float32)
         + jnp.dot(ef_ref[...], w1c_ref[...], preferred_element_type=jnp.float32)
         + b1_ref[...])
    h = 1.0 / (1.0 + jnp.exp(-x))
    out_ref[...] = (jnp.dot(h, w2_ref[...], preferred_element_type=jnp.float32)
                    + b2_ref[...])


def _tc_mlp(sg, rg, edges, w1a, w1b, w1c, b1, w2, b2):
    e_total, d_edge = edges.shape
    d_hidden = w2.shape[0]
    d_out = w2.shape[1]
    grid = (e_total // _BE,)
    return pl.pallas_call(
        _mlp_body,
        grid=grid,
        in_specs=[
            pl.BlockSpec((_BE, _D_NODE), lambda i: (i, 0)),
            pl.BlockSpec((_BE, _D_NODE), lambda i: (i, 0)),
            pl.BlockSpec((_BE, d_edge), lambda i: (i, 0)),
            pl.BlockSpec((_D_NODE, d_hidden), lambda i: (0, 0)),
            pl.BlockSpec((_D_NODE, d_hidden), lambda i: (0, 0)),
            pl.BlockSpec((d_edge, d_hidden), lambda i: (0, 0)),
            pl.BlockSpec((1, d_hidden), lambda i: (0, 0)),
            pl.BlockSpec((d_hidden, d_out), lambda i: (0, 0)),
            pl.BlockSpec((1, d_out), lambda i: (0, 0)),
        ],
        out_specs=pl.BlockSpec((_BE, d_out), lambda i: (i, 0)),
        out_shape=jax.ShapeDtypeStruct((e_total, d_out), jnp.float32),
    )(sg, rg, edges, w1a, w1b, w1c, b1, w2, b2)


_NCHUNK = 5


@jax.jit
def kernel(nodes, edges, W1, b1, W2, b2, senders, receivers):
    senders = senders.astype(jnp.int32)
    receivers = receivers.astype(jnp.int32)
    e_total = senders.shape[0]
    ec = e_total // _NCHUNK
    assert ec % (_NW * 2 * _CH) == 0 and ec % _BE == 0

    d_node = nodes.shape[1]
    w1t = W1.T
    w1a = w1t[:d_node]
    w1b = w1t[d_node:2 * d_node]
    w1c = w1t[2 * d_node:]
    b1r = b1.reshape(1, -1)
    w2t = W2.T
    b2r = b2.reshape(1, -1)

    outs = []
    for k in range(_NCHUNK):
        sl = slice(k * ec, (k + 1) * ec)
        sg, rg = _sc_gather(nodes, senders[sl], receivers[sl])
        outs.append(_tc_mlp(sg, rg, edges[sl], w1a, w1b, w1c, b1r, w2t, b2r))
    return jnp.concatenate(outs, axis=0)
